# baseline (speedup 1.0000x reference)
import jax
import jax.numpy as jnp
from jax.experimental import pallas as pl
from jax.experimental.pallas import tpu as pltpu

B, N, H = 16384, 1024, 256
TILE_B = 1024
LOG2E = 1.4426950408889634


def _retrieval_kernel(z_ref, mem_ref, out_ref, key_ref, val_ref):
    @pl.when(pl.program_id(0) == 0)
    def _prep():
        mem = mem_ref[...]
        m_inv = jax.lax.rsqrt(jnp.maximum(jnp.sum(mem * mem, axis=1, keepdims=True), 1e-24))
        key_ref[...] = (mem * (m_inv * LOG2E)).astype(jnp.bfloat16)
        val_ref[...] = mem.astype(jnp.bfloat16)

    z = z_ref[...]
    z_norm = z * jax.lax.rsqrt(jnp.maximum(jnp.sum(z * z, axis=1, keepdims=True), 1e-24))

    sim = jax.lax.dot_general(
        z_norm.astype(jnp.bfloat16), key_ref[...],
        (((1,), (1,)), ((), ())),
        preferred_element_type=jnp.float32,
    )

    e = jnp.exp2(sim.astype(jnp.bfloat16))

    ones_v = jnp.ones((N, 128), dtype=jnp.bfloat16)
    sums = jnp.dot(e, ones_v, preferred_element_type=jnp.float32)
    inv_sum = 1.0 / sums[:, 0:1]

    acc = jnp.dot(e, val_ref[...], preferred_element_type=jnp.float32)
    out_ref[...] = acc * inv_sum


def kernel(z, memory):
    return pl.pallas_call(
        _retrieval_kernel,
        grid=(B // TILE_B,),
        in_specs=[
            pl.BlockSpec((TILE_B, H), lambda i: (i, 0)),
            pl.BlockSpec((N, H), lambda i: (0, 0)),
        ],
        out_specs=pl.BlockSpec((TILE_B, H), lambda i: (i, 0)),
        out_shape=jax.ShapeDtypeStruct((B, H), jnp.float32),
        scratch_shapes=[
            pltpu.VMEM((N, H), jnp.bfloat16),
            pltpu.VMEM((N, H), jnp.bfloat16),
        ],
    )(z, memory)


# speedup vs baseline: 1.0073x; 1.0073x over previous
import jax
import jax.numpy as jnp
from jax.experimental import pallas as pl
from jax.experimental.pallas import tpu as pltpu

B, N, H = 16384, 1024, 256
TILE_B = 1024
HV = H + 128
LOG2E = 1.4426950408889634


def _retrieval_kernel(z_ref, mem_ref, out_ref, key_ref, val_ref):
    @pl.when(pl.program_id(0) == 0)
    def _prep():
        mem = mem_ref[...]
        m_inv = jax.lax.rsqrt(jnp.maximum(jnp.sum(mem * mem, axis=1, keepdims=True), 1e-24))
        key_ref[...] = (mem * (m_inv * LOG2E)).astype(jnp.bfloat16)
        val_ref[:, :H] = mem.astype(jnp.bfloat16)
        val_ref[:, H:] = jnp.ones((N, HV - H), dtype=jnp.bfloat16)

    z = z_ref[...]
    z_norm = z * jax.lax.rsqrt(jnp.maximum(jnp.sum(z * z, axis=1, keepdims=True), 1e-24))

    sim = jax.lax.dot_general(
        z_norm.astype(jnp.bfloat16), key_ref[...],
        (((1,), (1,)), ((), ())),
        preferred_element_type=jnp.float32,
    )

    e = jnp.exp2(sim.astype(jnp.bfloat16))

    acc = jnp.dot(e, val_ref[...], preferred_element_type=jnp.float32)
    out_ref[...] = acc[:, :H] * (1.0 / acc[:, H:H + 1])


def kernel(z, memory):
    return pl.pallas_call(
        _retrieval_kernel,
        grid=(B // TILE_B,),
        in_specs=[
            pl.BlockSpec((TILE_B, H), lambda i: (i, 0)),
            pl.BlockSpec((N, H), lambda i: (0, 0)),
        ],
        out_specs=pl.BlockSpec((TILE_B, H), lambda i: (i, 0)),
        out_shape=jax.ShapeDtypeStruct((B, H), jnp.float32),
        scratch_shapes=[
            pltpu.VMEM((N, H), jnp.bfloat16),
            pltpu.VMEM((N, HV), jnp.bfloat16),
        ],
    )(z, memory)
